# Initial kernel scaffold; baseline (speedup 1.0000x reference)
import jax
import jax.numpy as jnp
from jax.experimental import pallas as pl


def kernel(x, norm1_w, norm1_b, score_w, score_b, Wq, bq, Wkv, bkv, Wproj, bproj, norm2_w, norm2_b, W1, b1, W2, b2):
    raise NotImplementedError("write your pallas kernel here")


# speedup vs baseline: 4.9126x; 4.9126x over previous
import jax
import jax.numpy as jnp
from jax import lax
from jax.experimental import pallas as pl

B = 8
N = 2048
DIM = 192
S = 256
K = 32
H = 6
HD = DIM // H
MH = 768
SCALE = HD ** -0.5
F32 = jnp.float32
HIGHEST = lax.Precision.HIGHEST
BISECT_ITERS = 34
BR = 256
NB = N // BR


def _ln(x, w, b, eps=1e-5):
    mu = jnp.mean(x, axis=-1, keepdims=True)
    var = jnp.mean((x - mu) ** 2, axis=-1, keepdims=True)
    return (x - mu) / jnp.sqrt(var + eps) * w + b


def _prep_kernel(x_ref, n1w_ref, n1b_ref, sw_ref, sb_ref, Wkv_ref, bkv_ref,
                 xln_ref, ts_ref, kv_ref):
    xln = _ln(x_ref[0], n1w_ref[:], n1b_ref[:])
    xln_ref[0] = xln
    ts_ref[0] = jnp.dot(xln, sw_ref[:], preferred_element_type=F32) + sb_ref[0]
    kv_ref[0] = jnp.dot(xln, Wkv_ref[:], preferred_element_type=F32) + bkv_ref[:]


def _density_kernel(dmn_ref, noise_ref, dens_ref, rmax_ref):
    dmn = dmn_ref[0]
    val = dmn * dmn

    hi = jnp.max(val, axis=1, keepdims=True)
    rmax_ref[0] = jnp.max(dmn, axis=1, keepdims=True)
    lo = jnp.zeros((BR, 1), F32)

    def bisect(_, carry):
        lo_, hi_ = carry
        mid = 0.5 * (lo_ + hi_)
        cnt = jnp.sum((val <= mid).astype(F32), axis=1, keepdims=True)
        ge = cnt >= K
        return jnp.where(ge, lo_, mid), jnp.where(ge, mid, hi_)

    lo, hi = lax.fori_loop(0, BISECT_ITERS, bisect, (lo, hi))
    below = val < lo
    cnt_lt = jnp.sum(below.astype(F32), axis=1, keepdims=True)
    s_lt = jnp.sum(jnp.where(below, val, 0.0), axis=1, keepdims=True)
    ksum = s_lt + (K - cnt_lt) * hi
    dens_ref[0] = jnp.exp(-ksum * (1.0 / K)) + noise_ref[0]


def _score_kernel(dmn_ref, densr_ref, densc_ref, rmaxc_ref, score_ref):
    dmn = dmn_ref[0]
    dist_max = jnp.max(rmaxc_ref[0])
    masked = jnp.where(densc_ref[0] > densr_ref[0], dmn, jnp.inf)
    md = jnp.min(masked, axis=1, keepdims=True)
    dist = jnp.minimum(md, dist_max)
    score_ref[0] = dist * densr_ref[0]


def _rank_kernel(scr_ref, scc_ref, rank_ref):
    r = pl.program_id(1)
    jj = lax.broadcasted_iota(jnp.int32, (BR, N), 1)
    ii = lax.broadcasted_iota(jnp.int32, (BR, N), 0) + r * BR
    beats = (scc_ref[0] > scr_ref[0]) | ((scc_ref[0] == scr_ref[0]) & (jj < ii))
    rank_ref[0] = jnp.sum(beats.astype(F32), axis=1, keepdims=True)


def _select_kernel(rankc_ref, idx_ref):
    pr = lax.broadcasted_iota(jnp.int32, (S, N), 0).astype(F32)
    oh = (rankc_ref[0] == pr).astype(F32)
    nn = lax.broadcasted_iota(jnp.int32, (S, N), 1).astype(F32)
    idx_ref[0] = jnp.sum(oh * nn, axis=1, keepdims=True)


def _merge_kernel(xln_ref, rankc_ref, dmd_ref, ts_ref, Wq_ref, bq_ref,
                  merged_ref, q_ref):
    xln = xln_ref[0]
    rank_c = rankc_ref[0]
    dmd = dmd_ref[0]
    pr = lax.broadcasted_iota(jnp.int32, (S, N), 0).astype(F32)
    mn = jnp.min(dmd, axis=0, keepdims=True)
    idxm = jnp.min(jnp.where(dmd == mn, pr, float(S)), axis=0, keepdims=True)
    idx_cluster = jnp.where(rank_c < S, rank_c, idxm)

    oh2 = (idx_cluster == pr).astype(F32)
    tw = jnp.exp(ts_ref[0])
    aw = jnp.dot(oh2, tw, precision=HIGHEST,
                 preferred_element_type=F32) + 1e-6
    awt = lax.dot_general(oh2, aw, (((0,), (0,)), ((), ())),
                          precision=HIGHEST, preferred_element_type=F32)
    merged = jnp.dot(oh2, xln * (tw / awt), precision=HIGHEST,
                     preferred_element_type=F32)
    merged_ref[0] = merged
    q_ref[0] = jnp.dot(merged, Wq_ref[:], preferred_element_type=F32) + bq_ref[:]


def _attn_kernel(q_ref, k_ref, v_ref, tsc_ref, out_ref):
    logits = lax.dot_general(q_ref[0, 0], k_ref[0, 0], (((1,), (1,)), ((), ())),
                             preferred_element_type=F32) * SCALE + tsc_ref[0]
    m = jnp.max(logits, axis=1, keepdims=True)
    p = jnp.exp(logits - m)
    p = p / jnp.sum(p, axis=1, keepdims=True)
    out_ref[0, 0] = jnp.dot(p, v_ref[0, 0], preferred_element_type=F32)


def _epi_kernel(merged_ref, attn_ref, Wp_ref, bp_ref, n2w_ref, n2b_ref,
                W1_ref, b1_ref, W2_ref, b2_ref, out_ref):
    attn = jnp.dot(attn_ref[0], Wp_ref[:], preferred_element_type=F32) + bp_ref[:]
    feature = merged_ref[0] + attn
    h = _ln(feature, n2w_ref[:], n2b_ref[:])
    pre = jnp.dot(h, W1_ref[:], preferred_element_type=F32) + b1_ref[:]
    h = pre * 0.5 * (1.0 + lax.erf(pre * (2.0 ** -0.5)))
    out_ref[0] = feature + jnp.dot(h, W2_ref[:], preferred_element_type=F32) + b2_ref[:]


def _full(shape, ngrid):
    nd = len(shape)
    if ngrid == 1:
        return pl.BlockSpec(shape, lambda b, _n=nd: (0,) * _n)
    return pl.BlockSpec(shape, lambda b, r, _n=nd: (0,) * _n)


def kernel(x, norm1_w, norm1_b, score_w, score_b, Wq, bq, Wkv, bkv,
           Wproj, bproj, norm2_w, norm2_b, W1, b1, W2, b2):
    xf = _ln(x, norm1_w, norm1_b)
    sq = jnp.sum(xf * xf, axis=-1)
    d2 = sq[:, :, None] + sq[:, None, :] - 2.0 * jnp.einsum("bnd,bmd->bnm", xf, xf)
    dmn = jnp.sqrt(jnp.maximum(d2, 0.0)) / (DIM ** 0.5)

    noise = (jax.random.uniform(jax.random.key(42), (B, N), dtype=F32)
             * 1e-6).reshape(B, N, 1)

    xln, ts, kv = pl.pallas_call(
        _prep_kernel,
        grid=(B,),
        in_specs=[pl.BlockSpec((1, N, DIM), lambda b: (b, 0, 0))]
        + [_full(w.shape, 1) for w in (norm1_w, norm1_b, score_w, score_b, Wkv, bkv)],
        out_specs=[pl.BlockSpec((1, N, DIM), lambda b: (b, 0, 0)),
                   pl.BlockSpec((1, N, 1), lambda b: (b, 0, 0)),
                   pl.BlockSpec((1, N, 2 * DIM), lambda b: (b, 0, 0))],
        out_shape=[jax.ShapeDtypeStruct((B, N, DIM), F32),
                   jax.ShapeDtypeStruct((B, N, 1), F32),
                   jax.ShapeDtypeStruct((B, N, 2 * DIM), F32)],
    )(x, norm1_w, norm1_b, score_w, score_b, Wkv, bkv)

    rowd = pl.BlockSpec((1, BR, N), lambda b, r: (b, r, 0))
    row = pl.BlockSpec((1, BR, 1), lambda b, r: (b, r, 0))
    colv = pl.BlockSpec((1, 1, N), lambda b, r: (b, 0, 0))
    outrow = pl.BlockSpec((1, BR, 1), lambda b, r: (b, r, 0))

    density, rmax = pl.pallas_call(
        _density_kernel,
        grid=(B, NB),
        in_specs=[rowd, row],
        out_specs=[outrow, outrow],
        out_shape=[jax.ShapeDtypeStruct((B, N, 1), F32),
                   jax.ShapeDtypeStruct((B, N, 1), F32)],
    )(dmn, noise)

    score = pl.pallas_call(
        _score_kernel,
        grid=(B, NB),
        in_specs=[rowd, row, colv, colv],
        out_specs=outrow,
        out_shape=jax.ShapeDtypeStruct((B, N, 1), F32),
    )(dmn, density, density.reshape(B, 1, N), rmax.reshape(B, 1, N))

    rank = pl.pallas_call(
        _rank_kernel,
        grid=(B, NB),
        in_specs=[row, colv],
        out_specs=outrow,
        out_shape=jax.ShapeDtypeStruct((B, N, 1), F32),
    )(score, score.reshape(B, 1, N))
    rank_c = rank.reshape(B, 1, N)

    idx_f = pl.pallas_call(
        _select_kernel,
        grid=(B,),
        in_specs=[pl.BlockSpec((1, 1, N), lambda b: (b, 0, 0))],
        out_specs=pl.BlockSpec((1, S, 1), lambda b: (b, 0, 0)),
        out_shape=jax.ShapeDtypeStruct((B, S, 1), F32),
    )(rank_c)
    index_down = idx_f[..., 0].astype(jnp.int32)
    dmd = jnp.take_along_axis(
        dmn, index_down[:, :, None].astype(jnp.int32), axis=1)

    merged, q = pl.pallas_call(
        _merge_kernel,
        grid=(B,),
        in_specs=[pl.BlockSpec((1, N, DIM), lambda b: (b, 0, 0)),
                  pl.BlockSpec((1, 1, N), lambda b: (b, 0, 0)),
                  pl.BlockSpec((1, S, N), lambda b: (b, 0, 0)),
                  pl.BlockSpec((1, N, 1), lambda b: (b, 0, 0)),
                  _full(Wq.shape, 1), _full(bq.shape, 1)],
        out_specs=[pl.BlockSpec((1, S, DIM), lambda b: (b, 0, 0)),
                   pl.BlockSpec((1, S, DIM), lambda b: (b, 0, 0))],
        out_shape=[jax.ShapeDtypeStruct((B, S, DIM), F32),
                   jax.ShapeDtypeStruct((B, S, DIM), F32)],
    )(xln, rank_c, dmd, ts, Wq, bq)

    q4 = q.reshape(B, S, H, HD).transpose(0, 2, 1, 3)
    k4 = kv[..., :DIM].reshape(B, N, H, HD).transpose(0, 2, 1, 3)
    v4 = kv[..., DIM:].reshape(B, N, H, HD).transpose(0, 2, 1, 3)
    attn_h = pl.pallas_call(
        _attn_kernel,
        grid=(B, H),
        in_specs=[pl.BlockSpec((1, 1, S, HD), lambda b, h: (b, h, 0, 0)),
                  pl.BlockSpec((1, 1, N, HD), lambda b, h: (b, h, 0, 0)),
                  pl.BlockSpec((1, 1, N, HD), lambda b, h: (b, h, 0, 0)),
                  pl.BlockSpec((1, 1, N), lambda b, h: (b, 0, 0))],
        out_specs=pl.BlockSpec((1, 1, S, HD), lambda b, h: (b, h, 0, 0)),
        out_shape=jax.ShapeDtypeStruct((B, H, S, HD), F32),
    )(q4, k4, v4, ts.reshape(B, 1, N))

    attn = attn_h.transpose(0, 2, 1, 3).reshape(B, S, DIM)

    return pl.pallas_call(
        _epi_kernel,
        grid=(B,),
        in_specs=[pl.BlockSpec((1, S, DIM), lambda b: (b, 0, 0)),
                  pl.BlockSpec((1, S, DIM), lambda b: (b, 0, 0))]
        + [_full(w.shape, 1) for w in (Wproj, bproj, norm2_w, norm2_b, W1, b1, W2, b2)],
        out_specs=pl.BlockSpec((1, S, DIM), lambda b: (b, 0, 0)),
        out_shape=jax.ShapeDtypeStruct((B, S, DIM), F32),
    )(merged, attn, Wproj, bproj, norm2_w, norm2_b, W1, b1, W2, b2)
